# baseline (speedup 1.0000x reference)
import functools

import jax
import jax.numpy as jnp
from jax import lax
from jax.experimental import pallas as pl
from jax.experimental.pallas import tpu as pltpu
from jax.experimental.pallas import tpu_sc as plsc

_NC, _NS, _L = 2, 16, 16
_NW = _NC * _NS

_T, _K, _D = 16384, 9, 32
_R = _K - 1
_TW = _T // _NW
_CH = 64
_NCHUNK = _TW // _CH
_ROWS = _CH * _K
_SR = 128


def _body(ss_hbm, z_hbm, outq_hbm, outr_hbm, block_v, sid_v, g_v, q_v, o_v,
          sem):
    wid = lax.axis_index("s") * _NC + lax.axis_index("c")
    t0 = wid * _TW
    pltpu.sync_copy(ss_hbm.at[pl.ds(t0 * _K, _TW * _K)], block_v)

    lane = jnp.arange(16, dtype=jnp.int32)

    def sid_body(i, carry):
        ids = block_v[pl.ds(i * _L, _L)]
        sid_v[pl.ds(i * _L, _L)] = ids >> 2
        return carry

    lax.fori_loop(0, (_TW * _K) // _L, sid_body, 0)

    row_pat = 1 + (lane & 7)
    d_pat = [2 * v + (lane >> 3) for v in range(16)]

    for c in range(_NCHUNK):
        base = c * _ROWS
        cps = []
        for k in range(4):
            idx = sid_v.at[pl.ds(base + k * 128, 128)]
            cps.append(
                pltpu.async_copy(
                    z_hbm.at[idx], g_v.at[pl.ds(k * 128, 128)], sem
                )
            )
        idx = sid_v.at[pl.ds(base + 512, 64)]
        cps.append(
            pltpu.async_copy(z_hbm.at[idx], g_v.at[pl.ds(512, 64)], sem)
        )
        for cp in cps:
            cp.wait()

        def trial_body(tl, carry):
            g_base = tl * _K
            id_base = base + g_base
            rid = plsc.load_gather(block_v, [id_base + row_pat])
            qid = plsc.load_gather(block_v, [id_base + (lane & 0)])
            rcol = (rid & 3) * _D
            qcol = (qid & 3) * _D
            for v in range(_D // _L):
                vals = plsc.load_gather(
                    g_v, [g_base + (lane & 0), qcol + v * _L + lane]
                )
                q_v[pl.ds(tl * _D + v * _L, _L)] = vals
            for v in range(16):
                vals = plsc.load_gather(
                    g_v, [g_base + row_pat, rcol + d_pat[v]]
                )
                o_v[pl.ds(tl * (_D * _R) + v * _L, _L)] = vals
            return carry

        lax.fori_loop(0, _CH, trial_body, 0)

        tc0 = t0 + c * _CH
        pltpu.sync_copy(q_v, outq_hbm.at[pl.ds(tc0 * _D, _CH * _D)])
        pltpu.sync_copy(
            o_v, outr_hbm.at[pl.ds(tc0 * _D * _R, _CH * _D * _R)]
        )


@jax.jit
def _run(ss_flat, z4):
    kfn = pl.kernel(
        _body,
        out_type=(
            jax.ShapeDtypeStruct((_T * _D,), jnp.float32),
            jax.ShapeDtypeStruct((_T * _D * _R,), jnp.float32),
        ),
        mesh=plsc.VectorSubcoreMesh(
            core_axis_name="c", subcore_axis_name="s",
            num_cores=_NC, num_subcores=_NS,
        ),
        scratch_types=[
            pltpu.VMEM((_TW * _K,), jnp.int32),
            pltpu.VMEM((_TW * _K,), jnp.int32),
            pltpu.VMEM((_ROWS, _SR), jnp.float32),
            pltpu.VMEM((_CH * _D,), jnp.float32),
            pltpu.VMEM((_CH * _D * _R,), jnp.float32),
            pltpu.SemaphoreType.DMA,
        ],
        compiler_params=pltpu.CompilerParams(needs_layout_passes=False),
    )
    return kfn(ss_flat, z4)


def kernel(stimulus_set, max_n_reference, z):
    del max_n_reference
    z4 = z.reshape(z.shape[0] // 4, 4 * z.shape[1])
    q_flat, r_flat = _run(stimulus_set.reshape(-1), z4)
    return (
        q_flat.reshape(_T, _D, 1),
        r_flat.reshape(_T, _D, _R),
    )
